# Initial kernel scaffold; baseline (speedup 1.0000x reference)
import jax
import jax.numpy as jnp
from jax.experimental import pallas as pl


def kernel(atomic_numbers, coords, edge_index, batch, atom_table, fc_w1, fc_w2, readout_w, readout_b):
    raise NotImplementedError("write your pallas kernel here")


# speedup vs baseline: 4.6800x; 4.6800x over previous
import functools
import math

import jax
import jax.numpy as jnp
from jax import lax
from jax.experimental import pallas as pl
from jax.experimental.pallas import tpu as pltpu
from jax.experimental.pallas import tpu_sc as plsc

N_NODES = 10000
N_EDGES = 160000
NUM_ATOM_EMB = 100
ATOM_DIM = 16
HIDDEN = 16
EDGE_DIM = 16
CUTOFF = 6.0
N_GRAPHS = 64

NW = 32
E_PAD = 163840
EPT = E_PAD // NW
B_TC = 2048
N_ACC = 10240
DUMMY_ROW = N_NODES
N_PAD_TBL = 10240
FEAT = 4 * HIDDEN
CHUNK3 = 2048

_BES = 4.0 * math.sqrt(2.0 / CUTOFF)
_C0 = 0.25 * 0.25 * 0.25 / math.sqrt(4.0 * math.pi)
_C1 = _C0 * math.sqrt(3.0)


def _sc_mesh():
    return plsc.VectorSubcoreMesh(core_axis_name="c", subcore_axis_name="s")


def _gather_body(coords_hbm, atnum_hbm, src_hbm, dst_hbm,
                 vx_hbm, vy_hbm, vz_hbm, an_hbm,
                 coords_v, atnum_v, src_v, dst_v, vx_v, vy_v, vz_v, an_v):
    cid = lax.axis_index("c")
    sid = lax.axis_index("s")
    wid = sid * 2 + cid
    base = wid * EPT
    pltpu.sync_copy(coords_hbm, coords_v)
    pltpu.sync_copy(atnum_hbm, atnum_v)
    pltpu.sync_copy(src_hbm.at[pl.ds(base, EPT)], src_v)
    pltpu.sync_copy(dst_hbm.at[pl.ds(base, EPT)], dst_v)

    def body(i, carry):
        off = i * 16
        s = src_v[pl.ds(off, 16)]
        d = dst_v[pl.ds(off, 16)]
        s4 = s * 4
        d4 = d * 4
        vx_v[pl.ds(off, 16)] = (plsc.load_gather(coords_v, [s4])
                                - plsc.load_gather(coords_v, [d4]))
        vy_v[pl.ds(off, 16)] = (plsc.load_gather(coords_v, [s4 + 1])
                                - plsc.load_gather(coords_v, [d4 + 1]))
        vz_v[pl.ds(off, 16)] = (plsc.load_gather(coords_v, [s4 + 2])
                                - plsc.load_gather(coords_v, [d4 + 2]))
        an_v[pl.ds(off, 16)] = plsc.load_gather(atnum_v, [s])
        return carry

    lax.fori_loop(0, EPT // 16, body, 0)
    pltpu.sync_copy(vx_v, vx_hbm.at[pl.ds(base, EPT)])
    pltpu.sync_copy(vy_v, vy_hbm.at[pl.ds(base, EPT)])
    pltpu.sync_copy(vz_v, vz_hbm.at[pl.ds(base, EPT)])
    pltpu.sync_copy(an_v, an_hbm.at[pl.ds(base, EPT)])


def _gather_edges(coords_flat, atnum_pad, src_pad, dst_pad):
    return pl.kernel(
        _gather_body,
        out_type=(
            jax.ShapeDtypeStruct((E_PAD,), jnp.float32),
            jax.ShapeDtypeStruct((E_PAD,), jnp.float32),
            jax.ShapeDtypeStruct((E_PAD,), jnp.float32),
            jax.ShapeDtypeStruct((E_PAD,), jnp.int32),
        ),
        mesh=_sc_mesh(),
        compiler_params=pltpu.CompilerParams(needs_layout_passes=False),
        scratch_types=[
            pltpu.VMEM((N_PAD_TBL * 4,), jnp.float32),
            pltpu.VMEM((N_PAD_TBL,), jnp.int32),
            pltpu.VMEM((EPT,), jnp.int32),
            pltpu.VMEM((EPT,), jnp.int32),
            pltpu.VMEM((EPT,), jnp.float32),
            pltpu.VMEM((EPT,), jnp.float32),
            pltpu.VMEM((EPT,), jnp.float32),
            pltpu.VMEM((EPT,), jnp.int32),
        ],
    )(coords_flat, atnum_pad, src_pad, dst_pad)


def _dense_body(vx_ref, vy_ref, vz_ref, an_ref, atT_ref, w1T_ref, w01T_ref, msg_ref):
    vx = vx_ref[...]
    vy = vy_ref[...]
    vz = vz_ref[...]
    d2 = vx * vx + vy * vy + vz * vz + 1e-24
    dist = jnp.sqrt(d2)
    inv = 1.0 / dist
    ux = vx * inv
    uy = vy * inv
    uz = vz * inv
    dd = jnp.maximum(dist, 1e-6)
    freq = ((lax.broadcasted_iota(jnp.int32, (EDGE_DIM, 1), 0) + 1)
            .astype(jnp.float32) * (math.pi / CUTOFF))
    mask = (dist < CUTOFF).astype(jnp.float32)
    basis = (_BES * mask / dd) * jnp.sin(freq * dd)
    pre = jnp.dot(w1T_ref[...], basis, preferred_element_type=jnp.float32)
    hh = pre / (1.0 + jnp.exp(-pre))
    oh = (an_ref[...] == lax.broadcasted_iota(jnp.int32, (128, 1), 0)
          ).astype(jnp.float32)
    a = jnp.dot(atT_ref[...], oh, preferred_element_type=jnp.float32)
    outer = jnp.concatenate([a * hh[k:k + 1, :] for k in range(HIDDEN)], axis=0)
    big = jnp.dot(w01T_ref[...], outer, preferred_element_type=jnp.float32)
    b0 = big[:HIDDEN, :]
    b1 = big[HIDDEN:, :]
    msg_ref[...] = jnp.concatenate([b0, b1 * ux, b1 * uy, b1 * uz], axis=0)


def _dense_messages(vx, vy, vz, an, atableT, fc_w1T, w01T):
    grid = (E_PAD // B_TC,)
    row = pl.BlockSpec((1, B_TC), lambda i: (0, i))
    return pl.pallas_call(
        _dense_body,
        grid=grid,
        in_specs=[
            row, row, row, row,
            pl.BlockSpec((ATOM_DIM, 128), lambda i: (0, 0)),
            pl.BlockSpec((HIDDEN, EDGE_DIM), lambda i: (0, 0)),
            pl.BlockSpec((2 * HIDDEN, 256), lambda i: (0, 0)),
        ],
        out_specs=pl.BlockSpec((FEAT, B_TC), lambda i: (0, i)),
        out_shape=jax.ShapeDtypeStruct((FEAT, E_PAD), jnp.float32),
    )(vx.reshape(1, E_PAD), vy.reshape(1, E_PAD), vz.reshape(1, E_PAD),
      an.reshape(1, E_PAD), atableT, fc_w1T, w01T)


def _scatter_body(msg_hbm, dst_hbm, out_hbm, dst_v, m0_v, m1_v, acc0_v, acc1_v):
    cid = lax.axis_index("c")
    sid = lax.axis_index("s")
    wid = sid * 2 + cid
    f0 = wid * 2

    z = jnp.zeros((16,), jnp.float32)

    def zz(i, carry):
        acc0_v[pl.ds(i * 16, 16)] = z
        acc1_v[pl.ds(i * 16, 16)] = z
        return carry

    lax.fori_loop(0, N_ACC // 16, zz, 0)

    def chunk(c, carry):
        e0 = c * CHUNK3
        pltpu.sync_copy(dst_hbm.at[pl.ds(e0, CHUNK3)], dst_v)
        pltpu.sync_copy(msg_hbm.at[pl.ds(f0 * E_PAD + e0, CHUNK3)], m0_v)
        pltpu.sync_copy(msg_hbm.at[pl.ds((f0 + 1) * E_PAD + e0, CHUNK3)], m1_v)

        def grp(i, cy):
            off = i * 16
            ix = dst_v[pl.ds(off, 16)]
            plsc.addupdate_scatter(acc0_v, [ix], m0_v[pl.ds(off, 16)])
            plsc.addupdate_scatter(acc1_v, [ix], m1_v[pl.ds(off, 16)])
            return cy

        lax.fori_loop(0, CHUNK3 // 16, grp, 0)
        return carry

    lax.fori_loop(0, E_PAD // CHUNK3, chunk, 0)
    pltpu.sync_copy(acc0_v, out_hbm.at[pl.ds(f0 * N_ACC, N_ACC)])
    pltpu.sync_copy(acc1_v, out_hbm.at[pl.ds((f0 + 1) * N_ACC, N_ACC)])


def _scatter_messages(msg_flat, dst_pad):
    return pl.kernel(
        _scatter_body,
        out_type=jax.ShapeDtypeStruct((FEAT * N_ACC,), jnp.float32),
        mesh=_sc_mesh(),
        compiler_params=pltpu.CompilerParams(needs_layout_passes=False),
        scratch_types=[
            pltpu.VMEM((CHUNK3,), jnp.int32),
            pltpu.VMEM((CHUNK3,), jnp.float32),
            pltpu.VMEM((CHUNK3,), jnp.float32),
            pltpu.VMEM((N_ACC,), jnp.float32),
            pltpu.VMEM((N_ACC,), jnp.float32),
        ],
    )(msg_flat, dst_pad)


def _readout_body(nt_ref, b_ref, w_ref, rb_ref, nf_ref, gz_ref):
    nt = nt_ref[...]
    nf_ref[...] = jnp.transpose(nt[:, :N_NODES], (1, 0))
    z = jnp.sum(nt[:HIDDEN, :] * w_ref[...], axis=0, keepdims=True) + rb_ref[...]
    oh = (b_ref[...] == lax.broadcasted_iota(jnp.int32, (N_GRAPHS, 1), 0)
          ).astype(jnp.float32)
    gz_ref[...] = jnp.sum(oh * z, axis=1, keepdims=True)


def _readout(nodeT, batch_pad, wcol, rb):
    return pl.pallas_call(
        _readout_body,
        out_shape=(
            jax.ShapeDtypeStruct((N_NODES, FEAT), jnp.float32),
            jax.ShapeDtypeStruct((N_GRAPHS, 1), jnp.float32),
        ),
    )(nodeT, batch_pad, wcol, rb)


def kernel(atomic_numbers, coords, edge_index, batch, atom_table, fc_w1, fc_w2,
           readout_w, readout_b):
    f32 = jnp.float32
    src = edge_index[0].astype(jnp.int32)
    dst = edge_index[1].astype(jnp.int32)
    n_pad = E_PAD - N_EDGES
    src_pad = jnp.concatenate([src, jnp.zeros((n_pad,), jnp.int32)])
    dst_pad = jnp.concatenate([dst, jnp.full((n_pad,), DUMMY_ROW, jnp.int32)])

    coords_flat = (jnp.zeros((N_PAD_TBL, 4), f32)
                   .at[:N_NODES, :3].set(coords.astype(f32)).reshape(-1))
    atnum_pad = jnp.zeros((N_PAD_TBL,), jnp.int32).at[:N_NODES].set(
        atomic_numbers.astype(jnp.int32))

    fc_w1T = jnp.transpose(fc_w1.astype(f32) * 0.25)
    w2r = fc_w2.astype(f32).reshape(HIDDEN, 2, ATOM_DIM, HIDDEN)
    w0r = w2r[:, 0].reshape(HIDDEN * ATOM_DIM, HIDDEN) * _C0
    w1r = w2r[:, 1].reshape(HIDDEN * ATOM_DIM, HIDDEN) * _C1
    w01T = jnp.transpose(jnp.concatenate([w0r, w1r], axis=1))
    atableT = jnp.transpose(jnp.zeros((128, ATOM_DIM), f32)
                            .at[:NUM_ATOM_EMB].set(atom_table.astype(f32)))

    vx, vy, vz, an = _gather_edges(coords_flat, atnum_pad, src_pad, dst_pad)
    msgT = _dense_messages(vx, vy, vz, an, atableT, fc_w1T, w01T)
    node_flat = _scatter_messages(msgT.reshape(-1), dst_pad)

    batch_pad = jnp.concatenate([batch.astype(jnp.int32),
                                 jnp.full((N_ACC - N_NODES,), N_GRAPHS, jnp.int32)])
    nf, gz = _readout(node_flat.reshape(FEAT, N_ACC),
                      batch_pad.reshape(1, N_ACC),
                      readout_w.astype(f32).reshape(HIDDEN, 1),
                      readout_b.astype(f32).reshape(1, 1))
    return gz, nf


# speedup vs baseline: 6.9203x; 1.4787x over previous
import functools
import math

import jax
import jax.numpy as jnp
from jax import lax
from jax.experimental import pallas as pl
from jax.experimental.pallas import tpu as pltpu
from jax.experimental.pallas import tpu_sc as plsc

N_NODES = 10000
N_EDGES = 160000
NUM_ATOM_EMB = 100
ATOM_DIM = 16
HIDDEN = 16
EDGE_DIM = 16
CUTOFF = 6.0
N_GRAPHS = 64

NW = 32
E_PAD = 163840
EPT = E_PAD // NW
B_TC = 2048
N_ACC = 10240
DUMMY_ROW = N_NODES
N_PAD_TBL = 10240
FEAT = 4 * HIDDEN
CHUNK3 = 4096

_BES = 4.0 * math.sqrt(2.0 / CUTOFF)
_C0 = 0.25 * 0.25 * 0.25 / math.sqrt(4.0 * math.pi)
_C1 = _C0 * math.sqrt(3.0)


def _sc_mesh():
    return plsc.VectorSubcoreMesh(core_axis_name="c", subcore_axis_name="s")


def _gather_body(coords_hbm, atnum_hbm, src_hbm, dst_hbm,
                 vx_hbm, vy_hbm, vz_hbm, an_hbm,
                 coords_v, atnum_v, src_v, dst_v, vx_v, vy_v, vz_v, an_v):
    cid = lax.axis_index("c")
    sid = lax.axis_index("s")
    wid = sid * 2 + cid
    base = wid * EPT
    pltpu.sync_copy(coords_hbm, coords_v)
    pltpu.sync_copy(atnum_hbm, atnum_v)
    pltpu.sync_copy(src_hbm.at[pl.ds(base, EPT)], src_v)
    pltpu.sync_copy(dst_hbm.at[pl.ds(base, EPT)], dst_v)

    def body(i, carry):
        off = i * 16
        s = src_v[pl.ds(off, 16)]
        d = dst_v[pl.ds(off, 16)]
        s4 = s * 4
        d4 = d * 4
        vx_v[pl.ds(off, 16)] = (plsc.load_gather(coords_v, [s4])
                                - plsc.load_gather(coords_v, [d4]))
        vy_v[pl.ds(off, 16)] = (plsc.load_gather(coords_v, [s4 + 1])
                                - plsc.load_gather(coords_v, [d4 + 1]))
        vz_v[pl.ds(off, 16)] = (plsc.load_gather(coords_v, [s4 + 2])
                                - plsc.load_gather(coords_v, [d4 + 2]))
        an_v[pl.ds(off, 16)] = plsc.load_gather(atnum_v, [s])
        return carry

    lax.fori_loop(0, EPT // 16, body, 0)
    pltpu.sync_copy(vx_v, vx_hbm.at[pl.ds(base, EPT)])
    pltpu.sync_copy(vy_v, vy_hbm.at[pl.ds(base, EPT)])
    pltpu.sync_copy(vz_v, vz_hbm.at[pl.ds(base, EPT)])
    pltpu.sync_copy(an_v, an_hbm.at[pl.ds(base, EPT)])


def _gather_edges(coords_flat, atnum_pad, src_pad, dst_pad):
    return pl.kernel(
        _gather_body,
        out_type=(
            jax.ShapeDtypeStruct((E_PAD,), jnp.float32),
            jax.ShapeDtypeStruct((E_PAD,), jnp.float32),
            jax.ShapeDtypeStruct((E_PAD,), jnp.float32),
            jax.ShapeDtypeStruct((E_PAD,), jnp.int32),
        ),
        mesh=_sc_mesh(),
        compiler_params=pltpu.CompilerParams(needs_layout_passes=False),
        scratch_types=[
            pltpu.VMEM((N_PAD_TBL * 4,), jnp.float32),
            pltpu.VMEM((N_PAD_TBL,), jnp.int32),
            pltpu.VMEM((EPT,), jnp.int32),
            pltpu.VMEM((EPT,), jnp.int32),
            pltpu.VMEM((EPT,), jnp.float32),
            pltpu.VMEM((EPT,), jnp.float32),
            pltpu.VMEM((EPT,), jnp.float32),
            pltpu.VMEM((EPT,), jnp.int32),
        ],
    )(coords_flat, atnum_pad, src_pad, dst_pad)


def _dense_body(vx_ref, vy_ref, vz_ref, an_ref, atT_ref, w1T_ref, w01T_ref, msg_ref):
    vx = vx_ref[...]
    vy = vy_ref[...]
    vz = vz_ref[...]
    d2 = vx * vx + vy * vy + vz * vz + 1e-24
    dist = jnp.sqrt(d2)
    inv = 1.0 / dist
    ux = vx * inv
    uy = vy * inv
    uz = vz * inv
    dd = jnp.maximum(dist, 1e-6)
    freq = ((lax.broadcasted_iota(jnp.int32, (EDGE_DIM, 1), 0) + 1)
            .astype(jnp.float32) * (math.pi / CUTOFF))
    mask = (dist < CUTOFF).astype(jnp.float32)
    basis = (_BES * mask / dd) * jnp.sin(freq * dd)
    pre = jnp.dot(w1T_ref[...], basis, preferred_element_type=jnp.float32)
    hh = pre / (1.0 + jnp.exp(-pre))
    oh = (an_ref[...] == lax.broadcasted_iota(jnp.int32, (128, 1), 0)
          ).astype(jnp.float32)
    a = jnp.dot(atT_ref[...], oh, preferred_element_type=jnp.float32)
    outer = jnp.concatenate([a * hh[k:k + 1, :] for k in range(HIDDEN)], axis=0)
    big = jnp.dot(w01T_ref[...], outer, preferred_element_type=jnp.float32)
    b0 = big[:HIDDEN, :]
    b1 = big[HIDDEN:, :]
    msg_ref[...] = jnp.concatenate([b0, b1 * ux, b1 * uy, b1 * uz], axis=0)


def _dense_messages(vx, vy, vz, an, atableT, fc_w1T, w01T):
    grid = (E_PAD // B_TC,)
    row = pl.BlockSpec((1, B_TC), lambda i: (0, i))
    return pl.pallas_call(
        _dense_body,
        grid=grid,
        in_specs=[
            row, row, row, row,
            pl.BlockSpec((ATOM_DIM, 128), lambda i: (0, 0)),
            pl.BlockSpec((HIDDEN, EDGE_DIM), lambda i: (0, 0)),
            pl.BlockSpec((2 * HIDDEN, 256), lambda i: (0, 0)),
        ],
        out_specs=pl.BlockSpec((FEAT, B_TC), lambda i: (0, i)),
        out_shape=jax.ShapeDtypeStruct((FEAT, E_PAD), jnp.float32),
    )(vx.reshape(1, E_PAD), vy.reshape(1, E_PAD), vz.reshape(1, E_PAD),
      an.reshape(1, E_PAD), atableT, fc_w1T, w01T)


def _scatter_body(msg_hbm, dst_hbm, out_hbm,
                  dst0_v, m00_v, m01_v, dst1_v, m10_v, m11_v,
                  acc0_v, acc1_v, sem0, sem1):
    cid = lax.axis_index("c")
    sid = lax.axis_index("s")
    wid = sid * 2 + cid
    f0 = wid * 2
    nch = E_PAD // CHUNK3

    z = jnp.zeros((16,), jnp.float32)

    def zz(i, carry):
        acc0_v[pl.ds(i * 16, 16)] = z
        acc1_v[pl.ds(i * 16, 16)] = z
        return carry

    lax.fori_loop(0, N_ACC // 16, zz, 0)

    def start(c, dv, m0, m1, sem):
        e0 = c * CHUNK3
        pltpu.async_copy(dst_hbm.at[pl.ds(e0, CHUNK3)], dv, sem)
        pltpu.async_copy(msg_hbm.at[pl.ds(f0 * E_PAD + e0, CHUNK3)], m0, sem)
        pltpu.async_copy(msg_hbm.at[pl.ds((f0 + 1) * E_PAD + e0, CHUNK3)], m1, sem)

    def drain(dv, m0, m1, sem):
        pltpu.make_async_copy(dst_hbm.at[pl.ds(0, CHUNK3)], dv, sem).wait()
        pltpu.make_async_copy(msg_hbm.at[pl.ds(0, CHUNK3)], m0, sem).wait()
        pltpu.make_async_copy(msg_hbm.at[pl.ds(0, CHUNK3)], m1, sem).wait()

    def process(dv, m0, m1):
        def grp(i, cy):
            for u in range(4):
                off = i * 64 + u * 16
                ix = dv[pl.ds(off, 16)]
                plsc.addupdate_scatter(acc0_v, [ix], m0[pl.ds(off, 16)])
                plsc.addupdate_scatter(acc1_v, [ix], m1[pl.ds(off, 16)])
            return cy

        lax.fori_loop(0, CHUNK3 // 64, grp, 0)

    start(0, dst0_v, m00_v, m01_v, sem0)

    def body(i, carry):
        c0 = i * 2
        start(c0 + 1, dst1_v, m10_v, m11_v, sem1)
        drain(dst0_v, m00_v, m01_v, sem0)
        process(dst0_v, m00_v, m01_v)

        @pl.when(c0 + 2 < nch)
        def _():
            start(c0 + 2, dst0_v, m00_v, m01_v, sem0)

        drain(dst1_v, m10_v, m11_v, sem1)
        process(dst1_v, m10_v, m11_v)
        return carry

    lax.fori_loop(0, nch // 2, body, 0)
    pltpu.sync_copy(acc0_v, out_hbm.at[pl.ds(f0 * N_ACC, N_ACC)])
    pltpu.sync_copy(acc1_v, out_hbm.at[pl.ds((f0 + 1) * N_ACC, N_ACC)])


def _scatter_messages(msg_flat, dst_pad):
    return pl.kernel(
        _scatter_body,
        out_type=jax.ShapeDtypeStruct((FEAT * N_ACC,), jnp.float32),
        mesh=_sc_mesh(),
        compiler_params=pltpu.CompilerParams(needs_layout_passes=False),
        scratch_types=[
            pltpu.VMEM((CHUNK3,), jnp.int32),
            pltpu.VMEM((CHUNK3,), jnp.float32),
            pltpu.VMEM((CHUNK3,), jnp.float32),
            pltpu.VMEM((CHUNK3,), jnp.int32),
            pltpu.VMEM((CHUNK3,), jnp.float32),
            pltpu.VMEM((CHUNK3,), jnp.float32),
            pltpu.VMEM((N_ACC,), jnp.float32),
            pltpu.VMEM((N_ACC,), jnp.float32),
            pltpu.SemaphoreType.DMA,
            pltpu.SemaphoreType.DMA,
        ],
    )(msg_flat, dst_pad)


def _readout_body(nt_ref, b_ref, w_ref, rb_ref, nf_ref, gz_ref):
    nt = nt_ref[...]
    nf_ref[...] = jnp.transpose(nt[:, :N_NODES], (1, 0))
    z = jnp.sum(nt[:HIDDEN, :] * w_ref[...], axis=0, keepdims=True) + rb_ref[...]
    oh = (b_ref[...] == lax.broadcasted_iota(jnp.int32, (N_GRAPHS, 1), 0)
          ).astype(jnp.float32)
    gz_ref[...] = jnp.sum(oh * z, axis=1, keepdims=True)


def _readout(nodeT, batch_pad, wcol, rb):
    return pl.pallas_call(
        _readout_body,
        out_shape=(
            jax.ShapeDtypeStruct((N_NODES, FEAT), jnp.float32),
            jax.ShapeDtypeStruct((N_GRAPHS, 1), jnp.float32),
        ),
    )(nodeT, batch_pad, wcol, rb)


def kernel(atomic_numbers, coords, edge_index, batch, atom_table, fc_w1, fc_w2,
           readout_w, readout_b):
    f32 = jnp.float32
    src = edge_index[0].astype(jnp.int32)
    dst = edge_index[1].astype(jnp.int32)
    n_pad = E_PAD - N_EDGES
    src_pad = jnp.concatenate([src, jnp.zeros((n_pad,), jnp.int32)])
    dst_pad = jnp.concatenate([dst, jnp.full((n_pad,), DUMMY_ROW, jnp.int32)])

    coords_flat = (jnp.zeros((N_PAD_TBL, 4), f32)
                   .at[:N_NODES, :3].set(coords.astype(f32)).reshape(-1))
    atnum_pad = jnp.zeros((N_PAD_TBL,), jnp.int32).at[:N_NODES].set(
        atomic_numbers.astype(jnp.int32))

    fc_w1T = jnp.transpose(fc_w1.astype(f32) * 0.25)
    w2r = fc_w2.astype(f32).reshape(HIDDEN, 2, ATOM_DIM, HIDDEN)
    w0r = w2r[:, 0].reshape(HIDDEN * ATOM_DIM, HIDDEN) * _C0
    w1r = w2r[:, 1].reshape(HIDDEN * ATOM_DIM, HIDDEN) * _C1
    w01T = jnp.transpose(jnp.concatenate([w0r, w1r], axis=1))
    atableT = jnp.transpose(jnp.zeros((128, ATOM_DIM), f32)
                            .at[:NUM_ATOM_EMB].set(atom_table.astype(f32)))

    vx, vy, vz, an = _gather_edges(coords_flat, atnum_pad, src_pad, dst_pad)
    msgT = _dense_messages(vx, vy, vz, an, atableT, fc_w1T, w01T)
    node_flat = _scatter_messages(msgT.reshape(-1), dst_pad)

    batch_pad = jnp.concatenate([batch.astype(jnp.int32),
                                 jnp.full((N_ACC - N_NODES,), N_GRAPHS, jnp.int32)])
    nf, gz = _readout(node_flat.reshape(FEAT, N_ACC),
                      batch_pad.reshape(1, N_ACC),
                      readout_w.astype(f32).reshape(HIDDEN, 1),
                      readout_b.astype(f32).reshape(1, 1))
    return gz, nf


# speedup vs baseline: 7.2817x; 1.0522x over previous
import functools
import math

import jax
import jax.numpy as jnp
from jax import lax
from jax.experimental import pallas as pl
from jax.experimental.pallas import tpu as pltpu
from jax.experimental.pallas import tpu_sc as plsc

N_NODES = 10000
N_EDGES = 160000
NUM_ATOM_EMB = 100
ATOM_DIM = 16
HIDDEN = 16
EDGE_DIM = 16
CUTOFF = 6.0
N_GRAPHS = 64

NW = 32
E_PAD = 163840
EPT = E_PAD // NW
B_TC = 8192
N_ACC = 10240
DUMMY_ROW = N_NODES
N_PAD_TBL = 10240
FEAT = 4 * HIDDEN
CHUNK3 = 4096

_BES = 4.0 * math.sqrt(2.0 / CUTOFF)
_C0 = 0.25 * 0.25 * 0.25 / math.sqrt(4.0 * math.pi)
_C1 = _C0 * math.sqrt(3.0)


def _sc_mesh():
    return plsc.VectorSubcoreMesh(core_axis_name="c", subcore_axis_name="s")


def _gather_body(coords_hbm, atnum_hbm, src_hbm, dst_hbm,
                 vx_hbm, vy_hbm, vz_hbm, an_hbm,
                 coords_v, atnum_v, src_v, dst_v, vx_v, vy_v, vz_v, an_v):
    cid = lax.axis_index("c")
    sid = lax.axis_index("s")
    wid = sid * 2 + cid
    base = wid * EPT
    pltpu.sync_copy(coords_hbm, coords_v)
    pltpu.sync_copy(atnum_hbm, atnum_v)
    pltpu.sync_copy(src_hbm.at[pl.ds(base, EPT)], src_v)
    pltpu.sync_copy(dst_hbm.at[pl.ds(base, EPT)], dst_v)

    def body(i, carry):
        off = i * 16
        s = src_v[pl.ds(off, 16)]
        d = dst_v[pl.ds(off, 16)]
        s4 = s * 4
        d4 = d * 4
        vx_v[pl.ds(off, 16)] = (plsc.load_gather(coords_v, [s4])
                                - plsc.load_gather(coords_v, [d4]))
        vy_v[pl.ds(off, 16)] = (plsc.load_gather(coords_v, [s4 + 1])
                                - plsc.load_gather(coords_v, [d4 + 1]))
        vz_v[pl.ds(off, 16)] = (plsc.load_gather(coords_v, [s4 + 2])
                                - plsc.load_gather(coords_v, [d4 + 2]))
        an_v[pl.ds(off, 16)] = plsc.load_gather(atnum_v, [s])
        return carry

    lax.fori_loop(0, EPT // 16, body, 0)
    pltpu.sync_copy(vx_v, vx_hbm.at[pl.ds(base, EPT)])
    pltpu.sync_copy(vy_v, vy_hbm.at[pl.ds(base, EPT)])
    pltpu.sync_copy(vz_v, vz_hbm.at[pl.ds(base, EPT)])
    pltpu.sync_copy(an_v, an_hbm.at[pl.ds(base, EPT)])


def _gather_edges(coords_flat, atnum_pad, src_pad, dst_pad):
    return pl.kernel(
        _gather_body,
        out_type=(
            jax.ShapeDtypeStruct((E_PAD,), jnp.float32),
            jax.ShapeDtypeStruct((E_PAD,), jnp.float32),
            jax.ShapeDtypeStruct((E_PAD,), jnp.float32),
            jax.ShapeDtypeStruct((E_PAD,), jnp.int32),
        ),
        mesh=_sc_mesh(),
        compiler_params=pltpu.CompilerParams(needs_layout_passes=False),
        scratch_types=[
            pltpu.VMEM((N_PAD_TBL * 4,), jnp.float32),
            pltpu.VMEM((N_PAD_TBL,), jnp.int32),
            pltpu.VMEM((EPT,), jnp.int32),
            pltpu.VMEM((EPT,), jnp.int32),
            pltpu.VMEM((EPT,), jnp.float32),
            pltpu.VMEM((EPT,), jnp.float32),
            pltpu.VMEM((EPT,), jnp.float32),
            pltpu.VMEM((EPT,), jnp.int32),
        ],
    )(coords_flat, atnum_pad, src_pad, dst_pad)


def _dense_body(vx_ref, vy_ref, vz_ref, an_ref, atT_ref, w1T_ref, w01T_ref, msg_ref):
    vx = vx_ref[...]
    vy = vy_ref[...]
    vz = vz_ref[...]
    d2 = vx * vx + vy * vy + vz * vz + 1e-24
    dist = jnp.sqrt(d2)
    inv = 1.0 / dist
    ux = vx * inv
    uy = vy * inv
    uz = vz * inv
    dd = jnp.maximum(dist, 1e-6)
    freq = ((lax.broadcasted_iota(jnp.int32, (EDGE_DIM, 1), 0) + 1)
            .astype(jnp.float32) * (math.pi / CUTOFF))
    mask = (dist < CUTOFF).astype(jnp.float32)
    basis = (_BES * mask / dd) * jnp.sin(freq * dd)
    pre = jnp.dot(w1T_ref[...], basis, preferred_element_type=jnp.float32)
    hh = pre / (1.0 + jnp.exp(-pre))
    oh = (an_ref[...] == lax.broadcasted_iota(jnp.int32, (128, 1), 0)
          ).astype(jnp.float32)
    a = jnp.dot(atT_ref[...], oh, preferred_element_type=jnp.float32)
    outer = jnp.concatenate([a * hh[k:k + 1, :] for k in range(HIDDEN)], axis=0)
    big = jnp.dot(w01T_ref[...], outer, preferred_element_type=jnp.float32)
    b0 = big[:HIDDEN, :]
    b1 = big[HIDDEN:, :]
    msg_ref[...] = jnp.concatenate([b0, b1 * ux, b1 * uy, b1 * uz], axis=0)


def _dense_messages(vx, vy, vz, an, atableT, fc_w1T, w01T):
    grid = (E_PAD // B_TC,)
    row = pl.BlockSpec((1, B_TC), lambda i: (0, i))
    return pl.pallas_call(
        _dense_body,
        grid=grid,
        in_specs=[
            row, row, row, row,
            pl.BlockSpec((ATOM_DIM, 128), lambda i: (0, 0)),
            pl.BlockSpec((HIDDEN, EDGE_DIM), lambda i: (0, 0)),
            pl.BlockSpec((2 * HIDDEN, 256), lambda i: (0, 0)),
        ],
        out_specs=pl.BlockSpec((FEAT, B_TC), lambda i: (0, i)),
        out_shape=jax.ShapeDtypeStruct((FEAT, E_PAD), jnp.float32),
    )(vx.reshape(1, E_PAD), vy.reshape(1, E_PAD), vz.reshape(1, E_PAD),
      an.reshape(1, E_PAD), atableT, fc_w1T, w01T)


def _scatter_body(msg_hbm, dst_hbm, out_hbm,
                  dst0_v, m00_v, m01_v, dst1_v, m10_v, m11_v,
                  acc0_v, acc1_v, sem0, sem1):
    cid = lax.axis_index("c")
    sid = lax.axis_index("s")
    wid = sid * 2 + cid
    f0 = wid * 2
    nch = E_PAD // CHUNK3

    z = jnp.zeros((16,), jnp.float32)

    def zz(i, carry):
        acc0_v[pl.ds(i * 16, 16)] = z
        acc1_v[pl.ds(i * 16, 16)] = z
        return carry

    lax.fori_loop(0, N_ACC // 16, zz, 0)

    def start(c, dv, m0, m1, sem):
        e0 = c * CHUNK3
        pltpu.async_copy(dst_hbm.at[pl.ds(e0, CHUNK3)], dv, sem)
        pltpu.async_copy(msg_hbm.at[pl.ds(f0 * E_PAD + e0, CHUNK3)], m0, sem)
        pltpu.async_copy(msg_hbm.at[pl.ds((f0 + 1) * E_PAD + e0, CHUNK3)], m1, sem)

    def drain(dv, m0, m1, sem):
        pltpu.make_async_copy(dst_hbm.at[pl.ds(0, CHUNK3)], dv, sem).wait()
        pltpu.make_async_copy(msg_hbm.at[pl.ds(0, CHUNK3)], m0, sem).wait()
        pltpu.make_async_copy(msg_hbm.at[pl.ds(0, CHUNK3)], m1, sem).wait()

    def process(dv, m0, m1):
        def grp(i, cy):
            for u in range(4):
                off = i * 64 + u * 16
                ix = dv[pl.ds(off, 16)]
                plsc.addupdate_scatter(acc0_v, [ix], m0[pl.ds(off, 16)])
                plsc.addupdate_scatter(acc1_v, [ix], m1[pl.ds(off, 16)])
            return cy

        lax.fori_loop(0, CHUNK3 // 64, grp, 0)

    start(0, dst0_v, m00_v, m01_v, sem0)

    def body(i, carry):
        c0 = i * 2
        start(c0 + 1, dst1_v, m10_v, m11_v, sem1)
        drain(dst0_v, m00_v, m01_v, sem0)
        process(dst0_v, m00_v, m01_v)

        @pl.when(c0 + 2 < nch)
        def _():
            start(c0 + 2, dst0_v, m00_v, m01_v, sem0)

        drain(dst1_v, m10_v, m11_v, sem1)
        process(dst1_v, m10_v, m11_v)
        return carry

    lax.fori_loop(0, nch // 2, body, 0)
    pltpu.sync_copy(acc0_v, out_hbm.at[pl.ds(f0 * N_ACC, N_ACC)])
    pltpu.sync_copy(acc1_v, out_hbm.at[pl.ds((f0 + 1) * N_ACC, N_ACC)])


def _scatter_messages(msg_flat, dst_pad):
    return pl.kernel(
        _scatter_body,
        out_type=jax.ShapeDtypeStruct((FEAT * N_ACC,), jnp.float32),
        mesh=_sc_mesh(),
        compiler_params=pltpu.CompilerParams(needs_layout_passes=False),
        scratch_types=[
            pltpu.VMEM((CHUNK3,), jnp.int32),
            pltpu.VMEM((CHUNK3,), jnp.float32),
            pltpu.VMEM((CHUNK3,), jnp.float32),
            pltpu.VMEM((CHUNK3,), jnp.int32),
            pltpu.VMEM((CHUNK3,), jnp.float32),
            pltpu.VMEM((CHUNK3,), jnp.float32),
            pltpu.VMEM((N_ACC,), jnp.float32),
            pltpu.VMEM((N_ACC,), jnp.float32),
            pltpu.SemaphoreType.DMA,
            pltpu.SemaphoreType.DMA,
        ],
    )(msg_flat, dst_pad)


def _readout_body(nt_ref, b_ref, w_ref, rb_ref, nf_ref, gz_ref):
    nt = nt_ref[...]
    nf_ref[...] = jnp.transpose(nt[:, :N_NODES], (1, 0))
    z = jnp.sum(nt[:HIDDEN, :] * w_ref[...], axis=0, keepdims=True) + rb_ref[...]
    oh = (b_ref[...] == lax.broadcasted_iota(jnp.int32, (N_GRAPHS, 1), 0)
          ).astype(jnp.float32)
    gz_ref[...] = jnp.sum(oh * z, axis=1, keepdims=True)


def _readout(nodeT, batch_pad, wcol, rb):
    return pl.pallas_call(
        _readout_body,
        out_shape=(
            jax.ShapeDtypeStruct((N_NODES, FEAT), jnp.float32),
            jax.ShapeDtypeStruct((N_GRAPHS, 1), jnp.float32),
        ),
    )(nodeT, batch_pad, wcol, rb)


def kernel(atomic_numbers, coords, edge_index, batch, atom_table, fc_w1, fc_w2,
           readout_w, readout_b):
    f32 = jnp.float32
    src = edge_index[0].astype(jnp.int32)
    dst = edge_index[1].astype(jnp.int32)
    n_pad = E_PAD - N_EDGES
    src_pad = jnp.concatenate([src, jnp.zeros((n_pad,), jnp.int32)])
    dst_pad = jnp.concatenate([dst, jnp.full((n_pad,), DUMMY_ROW, jnp.int32)])

    coords_flat = (jnp.zeros((N_PAD_TBL, 4), f32)
                   .at[:N_NODES, :3].set(coords.astype(f32)).reshape(-1))
    atnum_pad = jnp.zeros((N_PAD_TBL,), jnp.int32).at[:N_NODES].set(
        atomic_numbers.astype(jnp.int32))

    fc_w1T = jnp.transpose(fc_w1.astype(f32) * 0.25)
    w2r = fc_w2.astype(f32).reshape(HIDDEN, 2, ATOM_DIM, HIDDEN)
    w0r = w2r[:, 0].reshape(HIDDEN * ATOM_DIM, HIDDEN) * _C0
    w1r = w2r[:, 1].reshape(HIDDEN * ATOM_DIM, HIDDEN) * _C1
    w01T = jnp.transpose(jnp.concatenate([w0r, w1r], axis=1))
    atableT = jnp.transpose(jnp.zeros((128, ATOM_DIM), f32)
                            .at[:NUM_ATOM_EMB].set(atom_table.astype(f32)))

    vx, vy, vz, an = _gather_edges(coords_flat, atnum_pad, src_pad, dst_pad)
    msgT = _dense_messages(vx, vy, vz, an, atableT, fc_w1T, w01T)
    node_flat = _scatter_messages(msgT.reshape(-1), dst_pad)

    batch_pad = jnp.concatenate([batch.astype(jnp.int32),
                                 jnp.full((N_ACC - N_NODES,), N_GRAPHS, jnp.int32)])
    nf, gz = _readout(node_flat.reshape(FEAT, N_ACC),
                      batch_pad.reshape(1, N_ACC),
                      readout_w.astype(f32).reshape(HIDDEN, 1),
                      readout_b.astype(f32).reshape(1, 1))
    return gz, nf
